# Initial kernel scaffold; baseline (speedup 1.0000x reference)
import jax
import jax.numpy as jnp
from jax.experimental import pallas as pl


def kernel(x, edge_index, params):
    raise NotImplementedError("write your pallas kernel here")


# speedup vs baseline: 4.7087x; 4.7087x over previous
import functools

import jax
import jax.numpy as jnp
from jax import lax
from jax.experimental import pallas as pl
from jax.experimental.pallas import tpu as pltpu
from jax.experimental.pallas import tpu_sc as plsc

N = 10000
NP = 10240
E = 160000
D_IN = 256
INNER = 1024
D_H = 512
FC = 4
CW = D_H // FC
NC, NS = 2, 16
NW = NC * NS
EPW = E // NW
B = 125
NB = EPW // B
ROWS_PER_TILE = NP // NS
ZB = 128
NZ = ROWS_PER_TILE // ZB

NBK = 1024
GRID = NP // NBK

_SC_MESH = plsc.VectorSubcoreMesh(core_axis_name="c", subcore_axis_name="s")


@functools.partial(
    pl.kernel,
    out_type=jax.ShapeDtypeStruct((NC, NP, CW), jnp.float32),
    mesh=_SC_MESH,
    scratch_types=[
        pltpu.VMEM((NB, B), jnp.int32),
        pltpu.VMEM((B, CW), jnp.float32),
        pltpu.VMEM((ZB, CW), jnp.float32),
        pltpu.VMEM_SHARED((NP, CW), jnp.float32),
    ],
)
def _sc_deg(dst_hbm, ones_hbm, zeros_hbm, out_hbm, dstv, onesv, zerov, shared):
    c = lax.axis_index("c")
    s = lax.axis_index("s")
    w = c * NS + s
    pltpu.sync_copy(dst_hbm.at[w], dstv)
    pltpu.sync_copy(ones_hbm, onesv)
    pltpu.sync_copy(zeros_hbm, zerov)
    base = s * ROWS_PER_TILE
    for z in range(NZ):
        pltpu.sync_copy(zerov, shared.at[pl.ds(base + z * ZB, ZB)])
    plsc.subcore_barrier()

    def body(b, carry):
        pltpu.sync_copy(onesv, shared.at[dstv.at[b]], add=True)
        return carry

    lax.fori_loop(0, NB, body, 0)
    plsc.subcore_barrier()
    for z in range(NZ):
        sl = pl.ds(base + z * ZB, ZB)
        pltpu.sync_copy(shared.at[sl], out_hbm.at[c].at[sl])


@functools.partial(
    pl.kernel,
    out_type=jax.ShapeDtypeStruct((NC, FC, NP, CW), jnp.float32),
    mesh=_SC_MESH,
    scratch_types=[
        pltpu.VMEM((NB, B), jnp.int32),
        pltpu.VMEM((NB, B), jnp.int32),
        pltpu.VMEM((B, CW), jnp.float32),
        pltpu.VMEM((ZB, CW), jnp.float32),
        pltpu.VMEM_SHARED((NP, CW), jnp.float32),
        pltpu.SemaphoreType.DMA,
    ],
)
def _sc_scatter(src_hbm, dst_hbm, f_hbm, zeros_hbm, out_hbm,
                srcv, dstv, rows, zerov, shared, sem):
    c = lax.axis_index("c")
    s = lax.axis_index("s")
    w = c * NS + s
    pltpu.sync_copy(src_hbm.at[w], srcv)
    pltpu.sync_copy(dst_hbm.at[w], dstv)
    pltpu.sync_copy(zeros_hbm, zerov)
    base = s * ROWS_PER_TILE
    for fc in range(FC):
        for z in range(NZ):
            pltpu.sync_copy(zerov, shared.at[pl.ds(base + z * ZB, ZB)])
        plsc.subcore_barrier()

        def body(b, carry):
            pltpu.async_copy(f_hbm.at[fc].at[srcv.at[b]], rows, sem).wait()
            pltpu.sync_copy(rows, shared.at[dstv.at[b]], add=True)
            return carry

        lax.fori_loop(0, NB, body, 0)
        plsc.subcore_barrier()
        for z in range(NZ):
            sl = pl.ds(base + z * ZB, ZB)
            pltpu.sync_copy(shared.at[sl], out_hbm.at[c].at[fc].at[sl])


def _ln(v, g, b):
    m = jnp.mean(v, axis=-1, keepdims=True)
    var = jnp.mean((v - m) ** 2, axis=-1, keepdims=True)
    return (v - m) * lax.rsqrt(var + 1e-5) * g + b


def _ff_body(x_ref, w1_ref, b1_ref, g1_ref, be1_ref, w2_ref, b2_ref,
             g2_ref, be2_ref, o_ref):
    h = jnp.dot(x_ref[...], w1_ref[...], preferred_element_type=jnp.float32)
    h = h + b1_ref[...]
    h = h * jax.nn.sigmoid(h)
    h = _ln(h, g1_ref[...], be1_ref[...])
    h = jnp.dot(h, w2_ref[...], preferred_element_type=jnp.float32)
    h = h + b2_ref[...]
    o_ref[...] = _ln(h, g2_ref[...], be2_ref[...])


def _tc_ff(x, p):
    full = lambda shape: pl.BlockSpec(shape, lambda i: (0,) * len(shape))
    return pl.pallas_call(
        _ff_body,
        grid=(GRID,),
        in_specs=[
            pl.BlockSpec((NBK, D_IN), lambda i: (i, 0)),
            full((D_IN, INNER)),
            full((1, INNER)), full((1, INNER)), full((1, INNER)),
            full((INNER, D_H)),
            full((1, D_H)), full((1, D_H)), full((1, D_H)),
        ],
        out_specs=pl.BlockSpec((NBK, D_H), lambda i: (i, 0)),
        out_shape=jax.ShapeDtypeStruct((NP, D_H), jnp.float32),
    )(x, p['W1'], p['b1'].reshape(1, -1), p['ln1_g'].reshape(1, -1),
      p['ln1_b'].reshape(1, -1), p['W2'], p['b2'].reshape(1, -1),
      p['ln2_g'].reshape(1, -1), p['ln2_b'].reshape(1, -1))


def _prep_body(deg_ref, h_ref, norm_ref, f_ref):
    deg = deg_ref[0, :, 0:1] + deg_ref[1, :, 0:1]
    norm = lax.rsqrt(jnp.maximum(deg, 1.0))
    norm_ref[...] = jnp.broadcast_to(norm, (NBK, CW))
    for fc in range(FC):
        f_ref[fc] = h_ref[:, fc * CW:(fc + 1) * CW] * norm


def _tc_prep(deg_parts, h):
    return pl.pallas_call(
        _prep_body,
        grid=(GRID,),
        in_specs=[
            pl.BlockSpec((NC, NBK, CW), lambda i: (0, i, 0)),
            pl.BlockSpec((NBK, D_H), lambda i: (i, 0)),
        ],
        out_specs=[
            pl.BlockSpec((NBK, CW), lambda i: (i, 0)),
            pl.BlockSpec((FC, NBK, CW), lambda i: (0, i, 0)),
        ],
        out_shape=[
            jax.ShapeDtypeStruct((NP, CW), jnp.float32),
            jax.ShapeDtypeStruct((FC, NP, CW), jnp.float32),
        ],
    )(deg_parts, h)


def _res_body(h_ref, w_ref, b_ref, o_ref):
    o_ref[...] = (jnp.dot(h_ref[...], w_ref[...],
                          preferred_element_type=jnp.float32) + b_ref[...])


def _tc_res(h, w, b):
    return pl.pallas_call(
        _res_body,
        grid=(GRID,),
        in_specs=[
            pl.BlockSpec((NBK, D_H), lambda i: (i, 0)),
            pl.BlockSpec((D_H, D_H), lambda i: (0, 0)),
            pl.BlockSpec((1, D_H), lambda i: (0, 0)),
        ],
        out_specs=pl.BlockSpec((NBK, D_H), lambda i: (i, 0)),
        out_shape=jax.ShapeDtypeStruct((NP, D_H), jnp.float32),
    )(h, w, b.reshape(1, -1))


def _gcn_hnew(part_ref, hres_ref, norm_ref, w_ref, b_ref):
    norm = norm_ref[:, 0:1]
    agg = jnp.concatenate(
        [part_ref[0, fc] + part_ref[1, fc] for fc in range(FC)], axis=-1)
    f = agg * norm
    return (jnp.dot(f, w_ref[...], preferred_element_type=jnp.float32)
            + b_ref[...] + hres_ref[...]), norm


def _gcn_body(part_ref, hres_ref, norm_ref, w_ref, b_ref, h_ref, f_ref):
    hn, norm = _gcn_hnew(part_ref, hres_ref, norm_ref, w_ref, b_ref)
    h_ref[...] = hn
    for fc in range(FC):
        f_ref[fc] = hn[:, fc * CW:(fc + 1) * CW] * norm


def _tc_gcn(part, hres, norm128, w, b):
    return pl.pallas_call(
        _gcn_body,
        grid=(GRID,),
        in_specs=[
            pl.BlockSpec((NC, FC, NBK, CW), lambda i: (0, 0, i, 0)),
            pl.BlockSpec((NBK, D_H), lambda i: (i, 0)),
            pl.BlockSpec((NBK, CW), lambda i: (i, 0)),
            pl.BlockSpec((D_H, D_H), lambda i: (0, 0)),
            pl.BlockSpec((1, D_H), lambda i: (0, 0)),
        ],
        out_specs=[
            pl.BlockSpec((NBK, D_H), lambda i: (i, 0)),
            pl.BlockSpec((FC, NBK, CW), lambda i: (0, i, 0)),
        ],
        out_shape=[
            jax.ShapeDtypeStruct((NP, D_H), jnp.float32),
            jax.ShapeDtypeStruct((FC, NP, CW), jnp.float32),
        ],
    )(part, hres, norm128, w, b.reshape(1, -1))


def _gcn_final_body(part_ref, hres_ref, norm_ref, w_ref, b_ref, o_ref):
    hn, _ = _gcn_hnew(part_ref, hres_ref, norm_ref, w_ref, b_ref)
    rid = (pl.program_id(0) * NBK
           + lax.broadcasted_iota(jnp.int32, (NBK, 1), 0))
    hn = jnp.where(rid < N, hn, -jnp.inf)

    @pl.when(pl.program_id(0) == 0)
    def _():
        o_ref[...] = jnp.full((1, D_H), -jnp.inf, dtype=jnp.float32)

    o_ref[...] = jnp.maximum(o_ref[...], jnp.max(hn, axis=0, keepdims=True))


def _tc_gcn_final(part, hres, norm128, w, b):
    return pl.pallas_call(
        _gcn_final_body,
        grid=(GRID,),
        in_specs=[
            pl.BlockSpec((NC, FC, NBK, CW), lambda i: (0, 0, i, 0)),
            pl.BlockSpec((NBK, D_H), lambda i: (i, 0)),
            pl.BlockSpec((NBK, CW), lambda i: (i, 0)),
            pl.BlockSpec((D_H, D_H), lambda i: (0, 0)),
            pl.BlockSpec((1, D_H), lambda i: (0, 0)),
        ],
        out_specs=pl.BlockSpec((1, D_H), lambda i: (0, 0)),
        out_shape=jax.ShapeDtypeStruct((1, D_H), jnp.float32),
    )(part, hres, norm128, w, b.reshape(1, -1))


def kernel(x, edge_index, params):
    ei = edge_index.astype(jnp.int32)
    src3 = ei[0].reshape(NW, NB, B)
    dst3 = ei[1].reshape(NW, NB, B)
    x = jnp.pad(x, ((0, NP - N), (0, 0)))
    ones128 = jnp.ones((B, CW), jnp.float32)
    zeros128 = jnp.zeros((ZB, CW), jnp.float32)

    deg_parts = _sc_deg(dst3, ones128, zeros128)
    h = _tc_ff(x, params)
    norm128, f = _tc_prep(deg_parts, h)
    out = None
    for i in range(4):
        hres = _tc_res(h, params['res%d_W' % i], params['res%d_b' % i])
        part = _sc_scatter(src3, dst3, f, zeros128)
        if i < 3:
            h, f = _tc_gcn(part, hres, norm128,
                           params['gcn%d_W' % i], params['gcn%d_b' % i])
        else:
            out = _tc_gcn_final(part, hres, norm128,
                                params['gcn%d_W' % i], params['gcn%d_b' % i])
    return out


# speedup vs baseline: 5.8400x; 1.2403x over previous
import functools

import jax
import jax.numpy as jnp
from jax import lax
from jax.experimental import pallas as pl
from jax.experimental.pallas import tpu as pltpu
from jax.experimental.pallas import tpu_sc as plsc

N = 10000
NP = 10240
E = 160000
D_IN = 256
INNER = 1024
D_H = 512
FC = 4
CW = D_H // FC
NC, NS = 2, 16
NW = NC * NS
EPW = E // NW
B = 125
NB = EPW // B
ROWS_PER_TILE = NP // NS
ZB = 128
NZ = ROWS_PER_TILE // ZB

NBK = 1024
GRID = NP // NBK

_SC_MESH = plsc.VectorSubcoreMesh(core_axis_name="c", subcore_axis_name="s")


@functools.partial(
    pl.kernel,
    out_type=jax.ShapeDtypeStruct((NC, NP, CW), jnp.float32),
    mesh=_SC_MESH,
    scratch_types=[
        pltpu.VMEM((NB, B), jnp.int32),
        pltpu.VMEM((B, CW), jnp.float32),
        pltpu.VMEM_SHARED((NP, CW), jnp.float32),
    ],
)
def _sc_deg(dst_hbm, ones_hbm, zeros_hbm, out_hbm, dstv, onesv, shared):
    c = lax.axis_index("c")
    s = lax.axis_index("s")
    w = c * NS + s
    pltpu.sync_copy(dst_hbm.at[w], dstv)
    pltpu.sync_copy(ones_hbm, onesv)
    base = s * ROWS_PER_TILE
    sl = pl.ds(base, ROWS_PER_TILE)
    pltpu.sync_copy(zeros_hbm, shared.at[sl])
    plsc.subcore_barrier()

    def body(b, carry):
        pltpu.sync_copy(onesv, shared.at[dstv.at[b]], add=True)
        return carry

    lax.fori_loop(0, NB, body, 0)
    plsc.subcore_barrier()
    pltpu.sync_copy(shared.at[sl], out_hbm.at[c].at[sl])


@functools.partial(
    pl.kernel,
    out_type=jax.ShapeDtypeStruct((NC, FC, NP, CW), jnp.float32),
    mesh=_SC_MESH,
    scratch_types=[
        pltpu.VMEM((NB, B), jnp.int32),
        pltpu.VMEM((NB, B), jnp.int32),
        pltpu.VMEM((B, CW), jnp.float32),
        pltpu.VMEM((B, CW), jnp.float32),
        pltpu.VMEM_SHARED((NP, CW), jnp.float32),
        pltpu.SemaphoreType.DMA,
        pltpu.SemaphoreType.DMA,
    ],
)
def _sc_scatter(src_hbm, dst_hbm, f_hbm, zeros_hbm, out_hbm,
                srcv, dstv, rows0, rows1, shared, sem0, sem1):
    c = lax.axis_index("c")
    s = lax.axis_index("s")
    w = c * NS + s
    pltpu.sync_copy(src_hbm.at[w], srcv)
    pltpu.sync_copy(dst_hbm.at[w], dstv)
    base = s * ROWS_PER_TILE
    stripe = pl.ds(base, ROWS_PER_TILE)
    rows = (rows0, rows1)
    sems = (sem0, sem1)
    for fc in range(FC):
        pltpu.sync_copy(zeros_hbm, shared.at[stripe])
        plsc.subcore_barrier()
        pltpu.async_copy(f_hbm.at[fc].at[srcv.at[0]], rows0, sem0)

        def body(i, carry):
            for j in range(2):
                b = 2 * i + j
                pltpu.make_async_copy(
                    f_hbm.at[fc].at[srcv.at[b]], rows[j], sems[j]).wait()

                @pl.when(b + 1 < NB)
                def _():
                    pltpu.async_copy(
                        f_hbm.at[fc].at[srcv.at[b + 1]], rows[j ^ 1],
                        sems[j ^ 1])

                pltpu.sync_copy(rows[j], shared.at[dstv.at[b]], add=True)
            return carry

        lax.fori_loop(0, NB // 2, body, 0)
        plsc.subcore_barrier()
        pltpu.sync_copy(shared.at[stripe], out_hbm.at[c].at[fc].at[stripe])


def _ln(v, g, b):
    m = jnp.mean(v, axis=-1, keepdims=True)
    var = jnp.mean((v - m) ** 2, axis=-1, keepdims=True)
    return (v - m) * lax.rsqrt(var + 1e-5) * g + b


def _ff_body(x_ref, w1_ref, b1_ref, g1_ref, be1_ref, w2_ref, b2_ref,
             g2_ref, be2_ref, o_ref):
    h = jnp.dot(x_ref[...], w1_ref[...], preferred_element_type=jnp.float32)
    h = h + b1_ref[...]
    h = h * jax.nn.sigmoid(h)
    h = _ln(h, g1_ref[...], be1_ref[...])
    h = jnp.dot(h, w2_ref[...], preferred_element_type=jnp.float32)
    h = h + b2_ref[...]
    o_ref[...] = _ln(h, g2_ref[...], be2_ref[...])


def _tc_ff(x, p):
    full = lambda shape: pl.BlockSpec(shape, lambda i: (0,) * len(shape))
    return pl.pallas_call(
        _ff_body,
        grid=(GRID,),
        in_specs=[
            pl.BlockSpec((NBK, D_IN), lambda i: (i, 0)),
            full((D_IN, INNER)),
            full((1, INNER)), full((1, INNER)), full((1, INNER)),
            full((INNER, D_H)),
            full((1, D_H)), full((1, D_H)), full((1, D_H)),
        ],
        out_specs=pl.BlockSpec((NBK, D_H), lambda i: (i, 0)),
        out_shape=jax.ShapeDtypeStruct((NP, D_H), jnp.float32),
    )(x, p['W1'], p['b1'].reshape(1, -1), p['ln1_g'].reshape(1, -1),
      p['ln1_b'].reshape(1, -1), p['W2'], p['b2'].reshape(1, -1),
      p['ln2_g'].reshape(1, -1), p['ln2_b'].reshape(1, -1))


def _prep_body(deg_ref, h_ref, norm_ref, f_ref):
    deg = deg_ref[0, :, 0:1] + deg_ref[1, :, 0:1]
    norm = lax.rsqrt(jnp.maximum(deg, 1.0))
    norm_ref[...] = jnp.broadcast_to(norm, (NBK, CW))
    for fc in range(FC):
        f_ref[fc] = h_ref[:, fc * CW:(fc + 1) * CW] * norm


def _tc_prep(deg_parts, h):
    return pl.pallas_call(
        _prep_body,
        grid=(GRID,),
        in_specs=[
            pl.BlockSpec((NC, NBK, CW), lambda i: (0, i, 0)),
            pl.BlockSpec((NBK, D_H), lambda i: (i, 0)),
        ],
        out_specs=[
            pl.BlockSpec((NBK, CW), lambda i: (i, 0)),
            pl.BlockSpec((FC, NBK, CW), lambda i: (0, i, 0)),
        ],
        out_shape=[
            jax.ShapeDtypeStruct((NP, CW), jnp.float32),
            jax.ShapeDtypeStruct((FC, NP, CW), jnp.float32),
        ],
    )(deg_parts, h)


def _res_body(h_ref, w_ref, b_ref, o_ref):
    o_ref[...] = (jnp.dot(h_ref[...], w_ref[...],
                          preferred_element_type=jnp.float32) + b_ref[...])


def _tc_res(h, w, b):
    return pl.pallas_call(
        _res_body,
        grid=(GRID,),
        in_specs=[
            pl.BlockSpec((NBK, D_H), lambda i: (i, 0)),
            pl.BlockSpec((D_H, D_H), lambda i: (0, 0)),
            pl.BlockSpec((1, D_H), lambda i: (0, 0)),
        ],
        out_specs=pl.BlockSpec((NBK, D_H), lambda i: (i, 0)),
        out_shape=jax.ShapeDtypeStruct((NP, D_H), jnp.float32),
    )(h, w, b.reshape(1, -1))


def _gcn_hnew(part_ref, hres_ref, norm_ref, w_ref, b_ref):
    norm = norm_ref[:, 0:1]
    agg = jnp.concatenate(
        [part_ref[0, fc] + part_ref[1, fc] for fc in range(FC)], axis=-1)
    f = agg * norm
    return (jnp.dot(f, w_ref[...], preferred_element_type=jnp.float32)
            + b_ref[...] + hres_ref[...]), norm


def _gcn_body(part_ref, hres_ref, norm_ref, w_ref, b_ref, h_ref, f_ref):
    hn, norm = _gcn_hnew(part_ref, hres_ref, norm_ref, w_ref, b_ref)
    h_ref[...] = hn
    for fc in range(FC):
        f_ref[fc] = hn[:, fc * CW:(fc + 1) * CW] * norm


def _tc_gcn(part, hres, norm128, w, b):
    return pl.pallas_call(
        _gcn_body,
        grid=(GRID,),
        in_specs=[
            pl.BlockSpec((NC, FC, NBK, CW), lambda i: (0, 0, i, 0)),
            pl.BlockSpec((NBK, D_H), lambda i: (i, 0)),
            pl.BlockSpec((NBK, CW), lambda i: (i, 0)),
            pl.BlockSpec((D_H, D_H), lambda i: (0, 0)),
            pl.BlockSpec((1, D_H), lambda i: (0, 0)),
        ],
        out_specs=[
            pl.BlockSpec((NBK, D_H), lambda i: (i, 0)),
            pl.BlockSpec((FC, NBK, CW), lambda i: (0, i, 0)),
        ],
        out_shape=[
            jax.ShapeDtypeStruct((NP, D_H), jnp.float32),
            jax.ShapeDtypeStruct((FC, NP, CW), jnp.float32),
        ],
    )(part, hres, norm128, w, b.reshape(1, -1))


def _gcn_final_body(part_ref, hres_ref, norm_ref, w_ref, b_ref, o_ref):
    hn, _ = _gcn_hnew(part_ref, hres_ref, norm_ref, w_ref, b_ref)
    rid = (pl.program_id(0) * NBK
           + lax.broadcasted_iota(jnp.int32, (NBK, 1), 0))
    hn = jnp.where(rid < N, hn, -jnp.inf)

    @pl.when(pl.program_id(0) == 0)
    def _():
        o_ref[...] = jnp.full((1, D_H), -jnp.inf, dtype=jnp.float32)

    o_ref[...] = jnp.maximum(o_ref[...], jnp.max(hn, axis=0, keepdims=True))


def _tc_gcn_final(part, hres, norm128, w, b):
    return pl.pallas_call(
        _gcn_final_body,
        grid=(GRID,),
        in_specs=[
            pl.BlockSpec((NC, FC, NBK, CW), lambda i: (0, 0, i, 0)),
            pl.BlockSpec((NBK, D_H), lambda i: (i, 0)),
            pl.BlockSpec((NBK, CW), lambda i: (i, 0)),
            pl.BlockSpec((D_H, D_H), lambda i: (0, 0)),
            pl.BlockSpec((1, D_H), lambda i: (0, 0)),
        ],
        out_specs=pl.BlockSpec((1, D_H), lambda i: (0, 0)),
        out_shape=jax.ShapeDtypeStruct((1, D_H), jnp.float32),
    )(part, hres, norm128, w, b.reshape(1, -1))


def kernel(x, edge_index, params):
    ei = edge_index.astype(jnp.int32)
    src3 = ei[0].reshape(NW, NB, B)
    dst3 = ei[1].reshape(NW, NB, B)
    x = jnp.pad(x, ((0, NP - N), (0, 0)))
    ones128 = jnp.ones((B, CW), jnp.float32)
    zeros128 = jnp.zeros((ROWS_PER_TILE, CW), jnp.float32)

    deg_parts = _sc_deg(dst3, ones128, zeros128)
    h = _tc_ff(x, params)
    norm128, f = _tc_prep(deg_parts, h)
    out = None
    for i in range(4):
        hres = _tc_res(h, params['res%d_W' % i], params['res%d_b' % i])
        part = _sc_scatter(src3, dst3, f, zeros128)
        if i < 3:
            h, f = _tc_gcn(part, hres, norm128,
                           params['gcn%d_W' % i], params['gcn%d_b' % i])
        else:
            out = _tc_gcn_final(part, hres, norm128,
                                params['gcn%d_W' % i], params['gcn%d_b' % i])
    return out
